# baseline (speedup 1.0000x reference)
import functools

import jax
import jax.numpy as jnp
from jax import lax
from jax.experimental import pallas as pl
from jax.experimental.pallas import tpu as pltpu
from jax.experimental.pallas import tpu_sc as plsc

IMG_W = 512.0
IMG_H = 512.0

_NC = 2
_NS = 16
_NW = _NC * _NS

_L_PER_W = 1024 // _NW
_GROUPS = _L_PER_W // 16
_C = 256
_P = 1024
_IDX_PER_W = _L_PER_W * _C
_ROWS = _IDX_PER_W // 128


def _splat(vec, i):
    idx = jnp.full((16, 1), i, jnp.int32)
    dnums = lax.GatherDimensionNumbers(
        offset_dims=(), collapsed_slice_dims=(0,), start_index_map=(0,))
    return lax.gather(vec, idx, dnums, (1,),
                      mode=lax.GatherScatterMode.PROMISE_IN_BOUNDS)


def _sc_body(flat_hbm, bbt_hbm, wt_hbm, b_hbm, out_hbm,
             bb_v, wt_v, b_v, idx_v, feat_v, out_v, sem):
    cell_w = jnp.float32(IMG_W / 32.0)
    cell_h = jnp.float32(IMG_H / 32.0)

    wid = lax.axis_index("s") * _NC + lax.axis_index("c")
    lbase = wid * _L_PER_W

    pltpu.sync_copy(bbt_hbm.at[wid], bb_v)
    pltpu.sync_copy(wt_hbm, wt_v)
    pltpu.sync_copy(b_hbm, b_v)

    iota = lax.iota(jnp.int32, 16)
    ramp = iota * _P

    base_g = []
    lab_g = []
    for g in range(_GROUPS):
        x = bb_v[0, pl.ds(g * 16, 16)]
        y = bb_v[1, pl.ds(g * 16, 16)]
        w = bb_v[2, pl.ds(g * 16, 16)]
        h = bb_v[3, pl.ds(g * 16, 16)]
        xc = x + (w / 2.0).astype(jnp.int32).astype(jnp.float32)
        yc = y + (h / 2.0).astype(jnp.int32).astype(jnp.float32)
        cx = (xc / cell_w).astype(jnp.int32)
        cy = (yc / cell_h).astype(jnp.int32)
        o = cy * 32 + cx
        lid = lbase + g * 16 + iota
        bi = lax.shift_right_logical(lid, 3)
        base_g.append(bi * (_C * _P) + o)

        cxf = cx.astype(jnp.float32)
        cyf = cy.astype(jnp.float32)
        lab_g.append(((xc - cxf * cell_w) / cell_w,
                      (yc - cyf * cell_h) / cell_h,
                      w / IMG_W,
                      h / IMG_H))

    for g in range(_GROUPS):
        bg = base_g[g]

        def _build(li, _, g=g, bg=bg):
            bsp = _splat(bg, li)
            l = g * 16 + li
            for t in range(_C // 16):
                idx_v[pl.ds(l * _C + t * 16, 16)] = bsp + (t * 16 * _P + ramp)
            return 0

        lax.fori_loop(0, 16, _build, 0)

    copies = []
    for j in range(_ROWS):
        copies.append(pltpu.async_copy(
            flat_hbm.at[idx_v.at[pl.ds(j * 128, 128)]],
            feat_v.at[pl.ds(j * 128, 128)], sem))
    for cp in copies:
        cp.wait()

    for g in range(_GROUPS):
        lx, ly, lw, lh = lab_g[g]

        def _fuse(li, _, g=g, lx=lx, ly=ly, lw=lw, lh=lh):
            l = g * 16 + li
            lxb = _splat(lx, li)
            lyb = _splat(ly, li)
            lwb = _splat(lw, li)
            lhb = _splat(lh, li)
            for t in range(_C // 16):
                c16 = pl.ds(t * 16, 16)
                acc = (feat_v[pl.ds(l * _C + t * 16, 16)]
                       + lxb * wt_v[0, c16] + lyb * wt_v[1, c16]
                       + lwb * wt_v[2, c16] + lhb * wt_v[3, c16]
                       + b_v[c16])
                out_v[pl.ds(l * _C + t * 16, 16)] = acc
            return 0

        lax.fori_loop(0, 16, _fuse, 0)

    pltpu.sync_copy(out_v, out_hbm.at[pl.ds(wid * _IDX_PER_W, _IDX_PER_W)])


def kernel(input, bboxes, W, b):
    B, K, N, _ = bboxes.shape
    C = input.shape[1]
    flat = input.reshape(-1)
    bbt = (bboxes.reshape(_NW, _L_PER_W, 4)
           .transpose(0, 2, 1).reshape(_NW, 4, _L_PER_W))
    wt = W.T

    mesh = plsc.VectorSubcoreMesh(core_axis_name="c", subcore_axis_name="s")
    run = functools.partial(
        pl.kernel, mesh=mesh,
        out_type=jax.ShapeDtypeStruct((B * K * N * C,), jnp.float32),
        scratch_types=[
            pltpu.VMEM((4, _L_PER_W), jnp.float32),
            pltpu.VMEM((4, _C), jnp.float32),
            pltpu.VMEM((_C,), jnp.float32),
            pltpu.VMEM((_IDX_PER_W,), jnp.int32),
            pltpu.VMEM((_IDX_PER_W,), jnp.float32),
            pltpu.VMEM((_IDX_PER_W,), jnp.float32),
            pltpu.SemaphoreType.DMA,
        ],
    )(_sc_body)
    out = run(flat, bbt, wt, b)
    return out.reshape(B, K, N, C)


# speedup vs baseline: 14.0471x; 14.0471x over previous
import functools

import jax
import jax.numpy as jnp
from jax import lax
from jax.experimental import pallas as pl
from jax.experimental.pallas import tpu as pltpu
from jax.experimental.pallas import tpu_sc as plsc

IMG_W = 512.0
IMG_H = 512.0

_NC = 2
_NS = 16
_NW = _NC * _NS

_L_PER_W = 1024 // _NW
_GROUPS = _L_PER_W // 16
_C = 256
_OUT_PER_W = _L_PER_W * _C


def _splat(vec, i):
    idx = jnp.full((16, 1), i, jnp.int32)
    dnums = lax.GatherDimensionNumbers(
        offset_dims=(), collapsed_slice_dims=(0,), start_index_map=(0,))
    return lax.gather(vec, idx, dnums, (1,),
                      mode=lax.GatherScatterMode.PROMISE_IN_BOUNDS)


def _sc_body(rows_hbm, bbt_hbm, wt_hbm, b_hbm, out_hbm,
             bb_v, wt_v, b_v, idx_v, feat_v, out_v, sem):
    cell_w = jnp.float32(IMG_W / 32.0)
    cell_h = jnp.float32(IMG_H / 32.0)

    wid = lax.axis_index("s") * _NC + lax.axis_index("c")

    pltpu.sync_copy(bbt_hbm.at[wid], bb_v)
    pltpu.sync_copy(wt_hbm, wt_v)
    pltpu.sync_copy(b_hbm, b_v)

    iota = lax.iota(jnp.int32, 16)

    lab_g = []
    for g in range(_GROUPS):
        x = bb_v[0, pl.ds(g * 16, 16)]
        y = bb_v[1, pl.ds(g * 16, 16)]
        w = bb_v[2, pl.ds(g * 16, 16)]
        h = bb_v[3, pl.ds(g * 16, 16)]
        xc = x + (w / 2.0).astype(jnp.int32).astype(jnp.float32)
        yc = y + (h / 2.0).astype(jnp.int32).astype(jnp.float32)
        cx = (xc / cell_w).astype(jnp.int32)
        cy = (yc / cell_h).astype(jnp.int32)
        lid = wid * _L_PER_W + g * 16 + iota
        bi = lax.shift_right_logical(lid, 3)
        r0 = ((bi * 32 + cy) * 64
              + lax.shift_right_logical(cx, 3) * 16 + (cx & 7))
        idx_v[pl.ds(g * 16, 16)] = r0
        idx_v[pl.ds(32 + g * 16, 16)] = r0 + 8

        cxf = cx.astype(jnp.float32)
        cyf = cy.astype(jnp.float32)
        lab_g.append(((xc - cxf * cell_w) / cell_w,
                      (yc - cyf * cell_h) / cell_h,
                      w / IMG_W,
                      h / IMG_H))

    pltpu.async_copy(rows_hbm.at[idx_v], feat_v, sem).wait()

    for g in range(_GROUPS):
        lx, ly, lw, lh = lab_g[g]

        def _fuse(li, _, g=g, lx=lx, ly=ly, lw=lw, lh=lh):
            l = g * 16 + li
            lxb = _splat(lx, li)
            lyb = _splat(ly, li)
            lwb = _splat(lw, li)
            lhb = _splat(lh, li)
            for t in range(_C // 16):
                half = t // 8
                col = (t % 8) * 16
                acc = (feat_v[half * _L_PER_W + l, pl.ds(col, 16)]
                       + lxb * wt_v[0, pl.ds(t * 16, 16)]
                       + lyb * wt_v[1, pl.ds(t * 16, 16)]
                       + lwb * wt_v[2, pl.ds(t * 16, 16)]
                       + lhb * wt_v[3, pl.ds(t * 16, 16)]
                       + b_v[pl.ds(t * 16, 16)])
                out_v[pl.ds(l * _C + t * 16, 16)] = acc
            return 0

        lax.fori_loop(0, 16, _fuse, 0)

    pltpu.sync_copy(out_v, out_hbm.at[pl.ds(wid * _OUT_PER_W, _OUT_PER_W)])


def kernel(input, bboxes, W, b):
    B, K, N, _ = bboxes.shape
    C = input.shape[1]
    fh, fw = input.shape[2], input.shape[3]
    rows = (input.transpose(0, 2, 3, 1)
            .reshape(B * K, fh, fw // 8, 8, C // 128, 128)
            .transpose(0, 1, 2, 4, 3, 5)
            .reshape(B * K * fh * (fw // 8) * (C // 128) * 8, 128))
    bbt = (bboxes.reshape(_NW, _L_PER_W, 4)
           .transpose(0, 2, 1).reshape(_NW, 4, _L_PER_W))
    wt = W.T

    mesh = plsc.VectorSubcoreMesh(core_axis_name="c", subcore_axis_name="s")
    run = functools.partial(
        pl.kernel, mesh=mesh,
        out_type=jax.ShapeDtypeStruct((B * K * N * C,), jnp.float32),
        scratch_types=[
            pltpu.VMEM((4, _L_PER_W), jnp.float32),
            pltpu.VMEM((4, _C), jnp.float32),
            pltpu.VMEM((_C,), jnp.float32),
            pltpu.VMEM((2 * _L_PER_W,), jnp.int32),
            pltpu.VMEM((2 * _L_PER_W, 128), jnp.float32),
            pltpu.VMEM((_OUT_PER_W,), jnp.float32),
            pltpu.SemaphoreType.DMA,
        ],
    )(_sc_body)
    out = run(rows, bbt, wt, b)
    return out.reshape(B, K, N, C)


# speedup vs baseline: 16.6417x; 1.1847x over previous
import functools

import jax
import jax.numpy as jnp
from jax import lax
from jax.experimental import pallas as pl
from jax.experimental.pallas import tpu as pltpu
from jax.experimental.pallas import tpu_sc as plsc

IMG_W = 512.0
IMG_H = 512.0

_NC = 2
_NS = 16
_NW = _NC * _NS

_L_PER_W = 1024 // _NW
_GROUPS = _L_PER_W // 16
_C = 256
_OUT_PER_W = _L_PER_W * _C


def _splat(vec, i):
    idx = jnp.full((16, 1), i, jnp.int32)
    dnums = lax.GatherDimensionNumbers(
        offset_dims=(), collapsed_slice_dims=(0,), start_index_map=(0,))
    return lax.gather(vec, idx, dnums, (1,),
                      mode=lax.GatherScatterMode.PROMISE_IN_BOUNDS)


def _sc_body(rows_hbm, bbt_hbm, wtb_hbm, out_hbm,
             bb_v, wtb_v, idx_v, feat_v, out_v, sem, sem2):
    cell_w = jnp.float32(IMG_W / 32.0)
    cell_h = jnp.float32(IMG_H / 32.0)

    wid = lax.axis_index("s") * _NC + lax.axis_index("c")

    pltpu.sync_copy(bbt_hbm.at[wid], bb_v)

    iota = lax.iota(jnp.int32, 16)

    lab_g = []
    for g in range(_GROUPS):
        x = bb_v[0, pl.ds(g * 16, 16)]
        y = bb_v[1, pl.ds(g * 16, 16)]
        w = bb_v[2, pl.ds(g * 16, 16)]
        h = bb_v[3, pl.ds(g * 16, 16)]
        xc = x + (w / 2.0).astype(jnp.int32).astype(jnp.float32)
        yc = y + (h / 2.0).astype(jnp.int32).astype(jnp.float32)
        cx = (xc / cell_w).astype(jnp.int32)
        cy = (yc / cell_h).astype(jnp.int32)
        lid = wid * _L_PER_W + g * 16 + iota
        bi = lax.shift_right_logical(lid, 3)
        r0 = ((bi * 32 + cy) * 64
              + lax.shift_right_logical(cx, 3) * 16 + (cx & 7))
        idx_v[pl.ds(g * 16, 16)] = r0
        idx_v[pl.ds(32 + g * 16, 16)] = r0 + 8

        cxf = cx.astype(jnp.float32)
        cyf = cy.astype(jnp.float32)
        lab_g.append(((xc - cxf * cell_w) / cell_w,
                      (yc - cyf * cell_h) / cell_h,
                      w / IMG_W,
                      h / IMG_H))

    copies = []
    for s in range(4):
        copies.append(pltpu.async_copy(
            rows_hbm.at[idx_v.at[pl.ds(s * 16, 16)]],
            feat_v.at[pl.ds(s * 16, 16)], sem))
    wtb_cp = pltpu.async_copy(wtb_hbm, wtb_v, sem2)
    wtb_cp.wait()
    for cp in copies:
        cp.wait()

    for half in range(2):
        wr = [[wtb_v[k, pl.ds(half * 128 + j * 16, 16)] for j in range(8)]
              for k in range(4)]
        br = [wtb_v[4, pl.ds(half * 128 + j * 16, 16)] for j in range(8)]
        for g in range(_GROUPS):
            lx, ly, lw, lh = lab_g[g]

            def _fuse(li, _, half=half, g=g, wr=wr, br=br,
                      lx=lx, ly=ly, lw=lw, lh=lh):
                l = g * 16 + li
                lxb = _splat(lx, li)
                lyb = _splat(ly, li)
                lwb = _splat(lw, li)
                lhb = _splat(lh, li)
                frow = half * _L_PER_W + l
                orow = (lax.shift_right_logical(l, 3) * 16
                        + half * 8 + (l & 7))
                for j in range(8):
                    acc = (feat_v[frow, pl.ds(j * 16, 16)]
                           + lxb * wr[0][j] + lyb * wr[1][j]
                           + lwb * wr[2][j] + lhb * wr[3][j] + br[j])
                    out_v[pl.ds(orow * 128 + j * 16, 16)] = acc
                return 0

            lax.fori_loop(0, 16, _fuse, 0)

    pltpu.sync_copy(out_v, out_hbm.at[pl.ds(wid * _OUT_PER_W, _OUT_PER_W)])


def kernel(input, bboxes, W, b):
    B, K, N, _ = bboxes.shape
    C = input.shape[1]
    fh, fw = input.shape[2], input.shape[3]
    rows = (input.transpose(0, 2, 3, 1)
            .reshape(B * K, fh, fw // 8, 8, C // 128, 128)
            .transpose(0, 1, 2, 4, 3, 5)
            .reshape(B * K * fh * (fw // 8) * (C // 128) * 8, 128))
    bbt = (bboxes.reshape(_NW, _L_PER_W, 4)
           .transpose(0, 2, 1).reshape(_NW, 4, _L_PER_W))
    wtb = jnp.concatenate([W.T, b.reshape(1, C)], axis=0)

    mesh = plsc.VectorSubcoreMesh(core_axis_name="c", subcore_axis_name="s")
    run = functools.partial(
        pl.kernel, mesh=mesh,
        out_type=jax.ShapeDtypeStruct((B * K * N * C,), jnp.float32),
        scratch_types=[
            pltpu.VMEM((4, _L_PER_W), jnp.float32),
            pltpu.VMEM((5, _C), jnp.float32),
            pltpu.VMEM((2 * _L_PER_W,), jnp.int32),
            pltpu.VMEM((2 * _L_PER_W, 128), jnp.float32),
            pltpu.VMEM((_OUT_PER_W,), jnp.float32),
            pltpu.SemaphoreType.DMA,
            pltpu.SemaphoreType.DMA,
        ],
    )(_sc_body)
    out = run(rows, bbt, wtb)
    return (out.reshape(B * K, C // 128, N, 128)
            .transpose(0, 2, 1, 3)
            .reshape(B, K, N, C))
